# baseline (speedup 1.0000x reference)
import functools

import jax
import jax.numpy as jnp
from jax.experimental import pallas as pl
from jax.experimental.pallas import tpu as pltpu

_A = 18
_ATOMS = 51
_AP = 64
_DH = 128
_NH = (1 + _A) * _AP
_NO = _A * _AP


def _fwd_kernel(x_ref, w123_ref, b123_ref, wq_ref, bq_ref, ind_ref, out_ref):
    h = x_ref[...]
    for l in range(3):
        z = jnp.dot(h, w123_ref[l], preferred_element_type=jnp.float32)
        h = jnp.maximum(z + b123_ref[l], 0.0)

    q = jnp.dot(h, wq_ref[...], preferred_element_type=jnp.float32)
    q = q + bq_ref[...]

    rmax = jnp.max(q, axis=-1, keepdims=True)
    e = jnp.exp(q - rmax)
    s = jnp.dot(e, ind_ref[...], preferred_element_type=jnp.float32)
    rrep = jnp.dot(1.0 / s, ind_ref[...].T,
                   preferred_element_type=jnp.float32)
    p = e * rrep
    out_ref[...] = p.reshape(p.shape[0], _A, _AP)[:, :, :_ATOMS]


@jax.jit
def _rainbow(x, w_pack, b_pack):
    B = x.shape[0]

    w123 = w_pack[:3, :_DH, :_DH]
    b123 = b_pack[:3, :, :_DH]
    w4 = w_pack[3, :_DH, :_NH]
    b4 = b_pack[3, 0, :_NH]
    wv = w4[:, :_AP]
    wa = w4[:, _AP:].reshape(_DH, _A, _AP)
    wq = (wv[:, None, :] + wa - wa.mean(1, keepdims=True)).reshape(_DH, _NO)
    bv = b4[:_AP]
    ba = b4[_AP:].reshape(_A, _AP)
    bq = (bv[None, :] + ba - ba.mean(0, keepdims=True)).reshape(1, _NO)
    lane = jnp.arange(_NO)
    bq = jnp.where((lane % _AP) < _ATOMS, bq, -1e30)
    ind = (lane[:, None] // _AP == jnp.arange(_A)[None, :]).astype(jnp.float32)

    bb = 512 if B % 512 == 0 else (128 if B % 128 == 0 else B)
    if B % bb:
        bpad = ((B + bb - 1) // bb) * bb
        x = jnp.zeros((bpad, x.shape[1]), x.dtype).at[:B].set(x)
    grid = (x.shape[0] // bb,)

    out = pl.pallas_call(
        _fwd_kernel,
        out_shape=jax.ShapeDtypeStruct((x.shape[0], _A, _ATOMS), jnp.float32),
        grid=grid,
        in_specs=[
            pl.BlockSpec((bb, _DH), lambda i: (i, 0)),
            pl.BlockSpec((3, _DH, _DH), lambda i: (0, 0, 0)),
            pl.BlockSpec((3, 1, _DH), lambda i: (0, 0, 0)),
            pl.BlockSpec((_DH, _NO), lambda i: (0, 0)),
            pl.BlockSpec((1, _NO), lambda i: (0, 0)),
            pl.BlockSpec((_NO, _A), lambda i: (0, 0)),
        ],
        out_specs=pl.BlockSpec((bb, _A, _ATOMS), lambda i: (i, 0, 0)),
        compiler_params=pltpu.CompilerParams(
            dimension_semantics=("parallel",)),
    )(x, w123, b123, wq, bq, ind)

    return out[:B]


def kernel(x, w_pack, b_pack):
    return _rainbow(x, w_pack, b_pack)


# speedup vs baseline: 1.2518x; 1.2518x over previous
import functools

import jax
import jax.numpy as jnp
from jax.experimental import pallas as pl
from jax.experimental.pallas import tpu as pltpu

_A = 18
_ATOMS = 51
_AP = 64
_DH = 128
_NH = (1 + _A) * _AP
_NO = _A * _AP


def _fwd_kernel(x_ref, w123_ref, b123_ref, w4_ref, b4_ref, mask_ref, ind_ref,
                out_ref):
    h = x_ref[...]
    for l in range(3):
        z = jnp.dot(h, w123_ref[l], preferred_element_type=jnp.float32)
        h = jnp.maximum(z + b123_ref[l], 0.0)

    w4 = w4_ref[0][:, :_NH]
    b4 = b4_ref[0][:, :_NH]
    wa = w4[:, _AP:]
    ba = b4[:, _AP:]
    ws = wa[:, 0:_AP]
    bs = ba[:, 0:_AP]
    for a in range(1, _A):
        ws = ws + wa[:, a * _AP:(a + 1) * _AP]
        bs = bs + ba[:, a * _AP:(a + 1) * _AP]
    wvm = w4[:, :_AP] - ws * (1.0 / _A)
    bvm = b4[:, :_AP] - bs * (1.0 / _A)
    wq = wa + jnp.concatenate([wvm] * _A, axis=1)
    bq = ba + jnp.concatenate([bvm] * _A, axis=1) + mask_ref[...]

    q = jnp.dot(h, wq, preferred_element_type=jnp.float32) + bq
    rmax = jnp.max(q, axis=-1, keepdims=True)
    e = jnp.exp(q - rmax)
    s = jnp.dot(e, ind_ref[...], preferred_element_type=jnp.float32)
    rrep = jnp.dot(1.0 / s, ind_ref[...].T,
                   preferred_element_type=jnp.float32)
    out_ref[...] = e * rrep


@jax.jit
def _rainbow(x, w_pack, b_pack):
    B = x.shape[0]

    lane = jnp.arange(_NO)
    mask = jnp.where((lane % _AP) < _ATOMS, 0.0, -1e30).reshape(1, _NO)
    ind = (lane[:, None] // _AP == jnp.arange(_A)[None, :]).astype(jnp.float32)

    bb = 512 if B % 512 == 0 else (128 if B % 128 == 0 else B)
    if B % bb:
        bpad = ((B + bb - 1) // bb) * bb
        x = jnp.zeros((bpad, x.shape[1]), x.dtype).at[:B].set(x)
    grid = (x.shape[0] // bb,)

    out = pl.pallas_call(
        _fwd_kernel,
        out_shape=jax.ShapeDtypeStruct((x.shape[0], _NO), jnp.float32),
        grid=grid,
        in_specs=[
            pl.BlockSpec((bb, _DH), lambda i: (i, 0)),
            pl.BlockSpec((3, _DH, _DH), lambda i: (0, 0, 0)),
            pl.BlockSpec((3, 1, _DH), lambda i: (0, 0, 0)),
            pl.BlockSpec((1, _DH, 1280), lambda i: (3, 0, 0)),
            pl.BlockSpec((1, 1, 1280), lambda i: (3, 0, 0)),
            pl.BlockSpec((1, _NO), lambda i: (0, 0)),
            pl.BlockSpec((_NO, _A), lambda i: (0, 0)),
        ],
        out_specs=pl.BlockSpec((bb, _NO), lambda i: (i, 0)),
        compiler_params=pltpu.CompilerParams(
            dimension_semantics=("parallel",)),
    )(x, w_pack, b_pack, w_pack, b_pack, mask, ind)

    return out[:B].reshape(B, _A, _AP)[:, :, :_ATOMS]


def kernel(x, w_pack, b_pack):
    return _rainbow(x, w_pack, b_pack)


# speedup vs baseline: 1.3580x; 1.0848x over previous
import functools

import jax
import jax.numpy as jnp
from jax.experimental import pallas as pl
from jax.experimental.pallas import tpu as pltpu

_A = 18
_ATOMS = 51
_AP = 64
_DH = 128
_NH = (1 + _A) * _AP
_NO = _A * _AP


def _fwd_kernel(x_ref, w123_ref, b123_ref, w4_ref, b4_ref, mask_ref, ind_ref,
                out_ref):
    h = x_ref[...]
    for l in range(3):
        z = jnp.dot(h, w123_ref[l], preferred_element_type=jnp.float32)
        h = jnp.maximum(z + b123_ref[l], 0.0)

    w4 = w4_ref[0][:, :_NH]
    b4 = b4_ref[0][:, :_NH]
    wa = w4[:, _AP:]
    ba = b4[:, _AP:]
    ws = wa[:, 0:_AP]
    bs = ba[:, 0:_AP]
    for a in range(1, _A):
        ws = ws + wa[:, a * _AP:(a + 1) * _AP]
        bs = bs + ba[:, a * _AP:(a + 1) * _AP]
    wvm = w4[:, :_AP] - ws * (1.0 / _A)
    bvm = b4[:, :_AP] - bs * (1.0 / _A)
    wq = wa + jnp.concatenate([wvm] * _A, axis=1)
    bq = ba + jnp.concatenate([bvm] * _A, axis=1) + mask_ref[...]

    q = jnp.dot(h, wq, preferred_element_type=jnp.float32) + bq
    rmax = jnp.max(q, axis=-1, keepdims=True)
    e = jnp.exp(q - rmax)
    s = jnp.dot(e, ind_ref[...], preferred_element_type=jnp.float32)
    rrep = jnp.dot(1.0 / s, ind_ref[...].T,
                   preferred_element_type=jnp.float32)
    out_ref[...] = e * rrep


@jax.jit
def _rainbow(x, w_pack, b_pack):
    B = x.shape[0]

    lane = jnp.arange(_NO)
    mask = jnp.where((lane % _AP) < _ATOMS, 0.0, -1e30).reshape(1, _NO)
    ind = (lane[:, None] // _AP == jnp.arange(_A)[None, :]).astype(jnp.float32)

    bb = 1024 if B % 1024 == 0 else (128 if B % 128 == 0 else B)
    if B % bb:
        bpad = ((B + bb - 1) // bb) * bb
        x = jnp.zeros((bpad, x.shape[1]), x.dtype).at[:B].set(x)
    grid = (x.shape[0] // bb,)

    out = pl.pallas_call(
        _fwd_kernel,
        out_shape=jax.ShapeDtypeStruct((x.shape[0], _NO), jnp.float32),
        grid=grid,
        in_specs=[
            pl.BlockSpec((bb, _DH), lambda i: (i, 0)),
            pl.BlockSpec((3, _DH, _DH), lambda i: (0, 0, 0)),
            pl.BlockSpec((3, 1, _DH), lambda i: (0, 0, 0)),
            pl.BlockSpec((1, _DH, 1280), lambda i: (3, 0, 0)),
            pl.BlockSpec((1, 1, 1280), lambda i: (3, 0, 0)),
            pl.BlockSpec((1, _NO), lambda i: (0, 0)),
            pl.BlockSpec((_NO, _A), lambda i: (0, 0)),
        ],
        out_specs=pl.BlockSpec((bb, _NO), lambda i: (i, 0)),
        compiler_params=pltpu.CompilerParams(
            dimension_semantics=("parallel",)),
    )(x, w_pack, b_pack, w_pack, b_pack, mask, ind)

    return out[:B].reshape(B, _A, _AP)[:, :, :_ATOMS]


def kernel(x, w_pack, b_pack):
    return _rainbow(x, w_pack, b_pack)


# speedup vs baseline: 1.3939x; 1.0264x over previous
import functools

import jax
import jax.numpy as jnp
from jax.experimental import pallas as pl
from jax.experimental.pallas import tpu as pltpu

_A = 18
_ATOMS = 51
_AP = 64
_DH = 128
_NH = (1 + _A) * _AP
_NO = _A * _AP


def _fwd_kernel(x_ref, w123_ref, b123_ref, w4_ref, b4_ref, mask_ref, ind_ref,
                out_ref):
    h = x_ref[...]
    for l in range(3):
        z = jnp.dot(h, w123_ref[l], preferred_element_type=jnp.float32)
        h = jnp.maximum(z + b123_ref[l], 0.0)

    w4 = w4_ref[0][:, :_NH]
    b4 = b4_ref[0][:, :_NH]
    wa = w4[:, _AP:]
    ba = b4[:, _AP:]
    ws = wa[:, 0:_AP]
    bs = ba[:, 0:_AP]
    for a in range(1, _A):
        ws = ws + wa[:, a * _AP:(a + 1) * _AP]
        bs = bs + ba[:, a * _AP:(a + 1) * _AP]
    wvm = w4[:, :_AP] - ws * (1.0 / _A)
    bvm = b4[:, :_AP] - bs * (1.0 / _A)
    wq = wa + jnp.concatenate([wvm] * _A, axis=1)
    bq = ba + jnp.concatenate([bvm] * _A, axis=1) + mask_ref[...]

    q = jnp.dot(h, wq, preferred_element_type=jnp.float32) + bq
    rmax = jnp.max(q, axis=-1, keepdims=True)
    e = jnp.exp(q - rmax)
    s = jnp.dot(e, ind_ref[...], preferred_element_type=jnp.float32)
    rrep = jnp.dot(1.0 / s, ind_ref[...].T,
                   preferred_element_type=jnp.float32)
    out_ref[...] = e * rrep


@jax.jit
def _rainbow(x, w_pack, b_pack):
    B = x.shape[0]

    lane = jnp.arange(_NO)
    mask = jnp.where((lane % _AP) < _ATOMS, 0.0, -1e30).reshape(1, _NO)
    ind = (lane[:, None] // _AP == jnp.arange(_A)[None, :]).astype(jnp.float32)

    bb = 2048 if B % 2048 == 0 else (128 if B % 128 == 0 else B)
    if B % bb:
        bpad = ((B + bb - 1) // bb) * bb
        x = jnp.zeros((bpad, x.shape[1]), x.dtype).at[:B].set(x)
    grid = (x.shape[0] // bb,)

    out = pl.pallas_call(
        _fwd_kernel,
        out_shape=jax.ShapeDtypeStruct((x.shape[0], _NO), jnp.float32),
        grid=grid,
        in_specs=[
            pl.BlockSpec((bb, _DH), lambda i: (i, 0)),
            pl.BlockSpec((3, _DH, _DH), lambda i: (0, 0, 0)),
            pl.BlockSpec((3, 1, _DH), lambda i: (0, 0, 0)),
            pl.BlockSpec((1, _DH, 1280), lambda i: (3, 0, 0)),
            pl.BlockSpec((1, 1, 1280), lambda i: (3, 0, 0)),
            pl.BlockSpec((1, _NO), lambda i: (0, 0)),
            pl.BlockSpec((_NO, _A), lambda i: (0, 0)),
        ],
        out_specs=pl.BlockSpec((bb, _NO), lambda i: (i, 0)),
        compiler_params=pltpu.CompilerParams(
            dimension_semantics=("parallel",)),
    )(x, w_pack, b_pack, w_pack, b_pack, mask, ind)

    return out[:B].reshape(B, _A, _AP)[:, :, :_ATOMS]


def kernel(x, w_pack, b_pack):
    return _rainbow(x, w_pack, b_pack)
